# baseline (speedup 1.0000x reference)
import functools

import jax
import jax.numpy as jnp
from jax import lax
from jax.experimental import pallas as pl
from jax.experimental.pallas import tpu as pltpu
from jax.experimental.pallas import tpu_sc as plsc

_N = 10000
_E = 320000
_D = 128
_A = 8
_B = _N // _A

_NC = 2
_NS = 16
_NW = _NC * _NS
_EPW = _E // _NW
_CH = 128
_NCHUNK = _EPW // _CH
_CHT = _EPW - _NCHUNK * _CH
_TOFF = _NCHUNK * _CH
_SPAN = 624
_ZR = 208
_ZRA = 48

_RB = 2000
_GB = 256
_GRID = _N // _RB


def _sc_mesh():
    return plsc.VectorSubcoreMesh(core_axis_name="c", subcore_axis_name="s")


@functools.partial(
    pl.kernel,
    mesh=_sc_mesh(),
    out_type=jax.ShapeDtypeStruct((_NC, _N, 16), jnp.float32),
    scratch_types=[
        pltpu.VMEM((_CH,), jnp.int32),
        pltpu.VMEM((_CH,), jnp.int32),
        pltpu.VMEM((_CH,), jnp.int32),
        pltpu.VMEM((_CH,), jnp.int32),
        pltpu.VMEM((_CHT,), jnp.int32),
        pltpu.VMEM((_CH, 16), jnp.float32),
        pltpu.VMEM((_ZR, 16), jnp.float32),
        pltpu.VMEM_SHARED((_N, 16), jnp.float32),
        pltpu.SemaphoreType.DMA,
        pltpu.SemaphoreType.DMA,
        pltpu.SemaphoreType.DMA,
        pltpu.SemaphoreType.DMA,
        pltpu.SemaphoreType.DMA,
        pltpu.SemaphoreType.DMA,
        pltpu.SemaphoreType.DMA,
        pltpu.SemaphoreType.DMA,
    ],
)
def _deg_kernel(dst_hbm, degp_hbm, di0, di1, di2, di3, dit_v, ones_v,
                zero_v, deg_sh, d0, d1, d2, d3, s0, s1, s2, s3):
    c = lax.axis_index("c")
    s = lax.axis_index("s")
    wid = c * _NS + s
    didx = (di0, di1, di2, di3)
    dsem = (d0, d1, d2, d3)
    ssem = (s0, s1, s2, s3)
    base = wid * _EPW

    def fire_i(j, k):
        off = pl.multiple_of(base + j * _CH, 8)
        pltpu.async_copy(dst_hbm.at[pl.ds(off, _CH)], didx[k], dsem[k])

    def wait_i(j, k):
        off = pl.multiple_of(base + j * _CH, 8)
        pltpu.make_async_copy(dst_hbm.at[pl.ds(off, _CH)], didx[k],
                              dsem[k]).wait()

    for t in range(4):
        fire_i(t, t)

    one16 = jnp.ones((16,), jnp.float32)
    zro16 = jnp.zeros((16,), jnp.float32)

    def fill_ones(i, _):
        ones_v[i, :] = one16
        return _

    lax.fori_loop(0, _CH, fill_ones, None)

    def fill_zero(i, _):
        zero_v[i, :] = zro16
        return _

    lax.fori_loop(0, _ZR, fill_zero, None)

    span = pl.multiple_of(s * _SPAN, 8)
    for z in range(3):
        pltpu.sync_copy(zero_v, deg_sh.at[pl.ds(span + z * _ZR, _ZR)])

    @pl.when(s == _NS - 1)
    def _zero_tail():
        pltpu.sync_copy(zero_v.at[pl.ds(0, 16)],
                        deg_sh.at[pl.ds(_NS * _SPAN, 16)])

    plsc.subcore_barrier()

    def body(i, _):
        c0 = i * 4
        for t in range(4):
            j = c0 + t
            wait_i(j, t)
            pltpu.sync_copy(ones_v, deg_sh.at[didx[t]], add=True)

            @pl.when(j + 4 < _NCHUNK)
            def _pref():
                fire_i(j + 4, t)
        return _

    lax.fori_loop(0, _NCHUNK // 4, body, None)
    for t in range(2):
        j = (_NCHUNK // 4) * 4 + t
        wait_i(j, t)
        pltpu.sync_copy(ones_v, deg_sh.at[didx[t]], add=True)

    toff = pl.multiple_of(base + _TOFF, 8)
    pltpu.sync_copy(dst_hbm.at[pl.ds(toff, _CHT)], dit_v)
    pltpu.sync_copy(ones_v.at[pl.ds(0, _CHT)], deg_sh.at[dit_v], add=True)
    plsc.subcore_barrier()

    pltpu.sync_copy(deg_sh.at[pl.ds(span, _SPAN)],
                    degp_hbm.at[c, pl.ds(span, _SPAN)])

    @pl.when(s == _NS - 1)
    def _copy_tail():
        pltpu.sync_copy(deg_sh.at[pl.ds(_NS * _SPAN, 16)],
                        degp_hbm.at[c, pl.ds(_NS * _SPAN, 16)])


@functools.partial(
    pl.kernel,
    mesh=_sc_mesh(),
    out_type=jax.ShapeDtypeStruct((_NC, _N, _D), jnp.float32),
    scratch_types=[
        pltpu.VMEM((_CH,), jnp.int32),
        pltpu.VMEM((_CH,), jnp.int32),
        pltpu.VMEM((_CH,), jnp.int32),
        pltpu.VMEM((_CH,), jnp.int32),
        pltpu.VMEM((_CH,), jnp.int32),
        pltpu.VMEM((_CH,), jnp.int32),
        pltpu.VMEM((_CH,), jnp.int32),
        pltpu.VMEM((_CH,), jnp.int32),
        pltpu.VMEM((_CHT,), jnp.int32),
        pltpu.VMEM((_CHT,), jnp.int32),
        pltpu.VMEM((_CH, _D), jnp.float32),
        pltpu.VMEM((_CH, _D), jnp.float32),
        pltpu.VMEM_SHARED((_N, _D), jnp.float32),
        pltpu.SemaphoreType.DMA,
        pltpu.SemaphoreType.DMA,
        pltpu.SemaphoreType.DMA,
        pltpu.SemaphoreType.DMA,
        pltpu.SemaphoreType.DMA,
        pltpu.SemaphoreType.DMA,
        pltpu.SemaphoreType.DMA,
        pltpu.SemaphoreType.DMA,
        pltpu.SemaphoreType.DMA,
        pltpu.SemaphoreType.DMA,
    ],
)
def _agg_kernel(src_hbm, dst_hbm, g_hbm, aggp_hbm,
                si0, si1, si2, si3, di0, di1, di2, di3, sit_v, dit_v,
                r0, r1, agg_sh,
                g0, g1, i0, i1, i2, i3, d0, d1, d2, d3):
    c = lax.axis_index("c")
    s = lax.axis_index("s")
    wid = c * _NS + s
    rows = (r0, r1)
    sidx = (si0, si1, si2, si3)
    didx = (di0, di1, di2, di3)
    gsem = (g0, g1)
    isem = (i0, i1, i2, i3)
    dsem = (d0, d1, d2, d3)
    base = wid * _EPW

    zro16 = jnp.zeros((16,), jnp.float32)

    def fill_zero(i, _):
        r = i // 8
        k = i % 8
        r0[r, pl.ds(k * 16, 16)] = zro16
        return _

    lax.fori_loop(0, _CH * 8, fill_zero, None)

    span = pl.multiple_of(s * _SPAN, 8)
    for z in range(_SPAN // _CH):
        pltpu.sync_copy(r0, agg_sh.at[pl.ds(span + z * _CH, _CH)])
    pltpu.sync_copy(r0.at[pl.ds(0, _SPAN % _CH)],
                    agg_sh.at[pl.ds(span + (_SPAN // _CH) * _CH,
                                    _SPAN % _CH)])

    @pl.when(s == _NS - 1)
    def _zero_tail():
        pltpu.sync_copy(r0.at[pl.ds(0, 16)],
                        agg_sh.at[pl.ds(_NS * _SPAN, 16)])

    plsc.subcore_barrier()

    def fire_i(j, m):
        off = pl.multiple_of(base + j * _CH, 8)
        pltpu.async_copy(src_hbm.at[pl.ds(off, _CH)], sidx[m], isem[m])
        pltpu.async_copy(dst_hbm.at[pl.ds(off, _CH)], didx[m], dsem[m])

    def wait_i(j, m):
        off = pl.multiple_of(base + j * _CH, 8)
        pltpu.make_async_copy(src_hbm.at[pl.ds(off, _CH)], sidx[m],
                              isem[m]).wait()
        pltpu.make_async_copy(dst_hbm.at[pl.ds(off, _CH)], didx[m],
                              dsem[m]).wait()

    def fire_g(j, m, k):
        pltpu.async_copy(g_hbm.at[sidx[m]], rows[k], gsem[k])

    def wait_g(m, k):
        pltpu.make_async_copy(g_hbm.at[sidx[m]], rows[k],
                              gsem[k]).wait()

    def sync_s(j, m, k):
        pltpu.sync_copy(rows[k], agg_sh.at[didx[m]], add=True)

    for t in range(4):
        fire_i(t, t)
    for t in range(2):
        wait_i(t, t)
        fire_g(t, t, t)

    def body(i, _):
        c0 = i * 4
        for t in range(4):
            c = c0 + t
            k = t % 2
            wait_g(t, k)
            sync_s(c, t, k)

            @pl.when(c + 4 < _NCHUNK)
            def _pref_i():
                fire_i(c + 4, t)

            @pl.when(c + 2 < _NCHUNK)
            def _pref_g():
                m2 = (t + 2) % 4
                wait_i(c + 2, m2)
                fire_g(c + 2, m2, k)
        return _

    lax.fori_loop(0, _NCHUNK // 4, body, None)
    for t in range(2):
        c = (_NCHUNK // 4) * 4 + t
        wait_g(t, t)
        sync_s(c, t, t)

    toff = pl.multiple_of(base + _TOFF, 8)
    pltpu.sync_copy(src_hbm.at[pl.ds(toff, _CHT)], sit_v)
    pltpu.sync_copy(dst_hbm.at[pl.ds(toff, _CHT)], dit_v)
    pltpu.async_copy(g_hbm.at[sit_v], r0.at[pl.ds(0, _CHT)], g0).wait()
    pltpu.sync_copy(r0.at[pl.ds(0, _CHT)], agg_sh.at[dit_v], add=True)
    plsc.subcore_barrier()

    pltpu.sync_copy(agg_sh.at[pl.ds(span, _SPAN)],
                    aggp_hbm.at[c, pl.ds(span, _SPAN)])

    @pl.when(s == _NS - 1)
    def _copy_tail():
        pltpu.sync_copy(agg_sh.at[pl.ds(_NS * _SPAN, 16)],
                        aggp_hbm.at[c, pl.ds(_NS * _SPAN, 16)])


def _h_body(state_ref, w_ref, h_ref):
    h_ref[...] = jnp.dot(state_ref[...], w_ref[...],
                         preferred_element_type=jnp.float32)


def _h_call(state, w_conv):
    return pl.pallas_call(
        _h_body,
        grid=(_GRID,),
        in_specs=[
            pl.BlockSpec((_RB, _D), lambda i: (i, 0)),
            pl.BlockSpec((_D, _D), lambda i: (0, 0)),
        ],
        out_specs=pl.BlockSpec((_RB, _D), lambda i: (i, 0)),
        out_shape=jax.ShapeDtypeStruct((_N, _D), jnp.float32),
    )(state, w_conv)


def _g_body(h_ref, degp_ref, g_ref):
    dp = degp_ref[0] + degp_ref[1]
    dinv = lax.rsqrt(dp[:, 0:1] + 1.0)
    g_ref[...] = h_ref[...] * dinv


def _g_call(h, degp):
    return pl.pallas_call(
        _g_body,
        grid=(_GRID,),
        in_specs=[
            pl.BlockSpec((_RB, _D), lambda i: (i, 0)),
            pl.BlockSpec((_NC, _RB, 16), lambda i: (0, i, 0)),
        ],
        out_specs=pl.BlockSpec((_RB, _D), lambda i: (i, 0)),
        out_shape=jax.ShapeDtypeStruct((_N, _D), jnp.float32),
    )(h, degp)


def _head_body(aggp_ref, g_ref, state_ref, degp_ref, acol_ref, bconv_ref,
               w1a_ref, w1b_ref, b1_ref, w2_ref, b2_ref, w3_ref, b3_ref,
               out_ref):
    agg = aggp_ref[0] + aggp_ref[1]
    dp = degp_ref[0] + degp_ref[1]
    dinv = lax.rsqrt(dp[:, 0:1] + 1.0)
    conv = dinv * (agg + g_ref[...]) + bconv_ref[...]
    x = jnp.maximum(conv, 0.0) + state_ref[...]
    y1 = jnp.dot(x, w1a_ref[...], preferred_element_type=jnp.float32)
    y1 = y1 + acol_ref[...] * w1b_ref[...] + b1_ref[...]
    y1 = jnp.maximum(y1, 0.0)
    y2 = jnp.dot(y1, w2_ref[...], preferred_element_type=jnp.float32)
    y2 = jnp.maximum(y2 + b2_ref[...], 0.0)
    rows = lax.broadcasted_iota(jnp.int32, (_GB, _RB), 0)
    cols = lax.broadcasted_iota(jnp.int32, (_GB, _RB), 1)
    sel = jnp.where((cols >> 3) == rows, 1.0, 0.0)
    grp = jnp.dot(sel, y2, preferred_element_type=jnp.float32)
    out_ref[...] = jnp.dot(grp, w3_ref[...],
                           preferred_element_type=jnp.float32) + b3_ref[...]


def _head_call(aggp, g, state, degp, acol, bconv, w1a, w1b, b1, w2, b2,
               w3p, b3p):
    full = lambda shape: pl.BlockSpec(shape, lambda i: tuple(0 for _ in shape))
    return pl.pallas_call(
        _head_body,
        grid=(_GRID,),
        in_specs=[
            pl.BlockSpec((_NC, _RB, _D), lambda i: (0, i, 0)),
            pl.BlockSpec((_RB, _D), lambda i: (i, 0)),
            pl.BlockSpec((_RB, _D), lambda i: (i, 0)),
            pl.BlockSpec((_NC, _RB, 16), lambda i: (0, i, 0)),
            pl.BlockSpec((_RB, 1), lambda i: (i, 0)),
            full((1, _D)),
            full((_D, 32)),
            full((1, 32)),
            full((1, 32)),
            full((32, 32)),
            full((1, 32)),
            full((32, _D)),
            full((1, _D)),
        ],
        out_specs=pl.BlockSpec((_GB, _D), lambda i: (i, 0)),
        out_shape=jax.ShapeDtypeStruct((_GRID * _GB, _D), jnp.float32),
    )(aggp, g, state, degp, acol, bconv, w1a, w1b, b1, w2, b2, w3p, b3p)


def kernel(state, edge_index, action, W_conv, b_conv, W1, b1, W2, b2, W3, b3):
    src = edge_index[0]
    dst = edge_index[1]

    h = _h_call(state, W_conv)
    degp = _deg_kernel(dst)
    g = _g_call(h, degp)
    aggp = _agg_kernel(src, dst, g)

    acol = action.reshape(_N, 1)
    w1a = W1[:_D]
    w1b = W1[_D:_D + 1]
    w3p = jnp.pad(W3, ((0, 0), (0, _D - 1)))
    b3p = jnp.pad(b3.reshape(1, 1), ((0, 0), (0, _D - 1)))

    out = _head_call(aggp, g, state, degp, acol, b_conv.reshape(1, _D),
                     w1a, w1b, b1.reshape(1, 32), W2, b2.reshape(1, 32),
                     w3p, b3p)

    return out[:, 0].reshape(_GRID, _GB)[:, :_B // _GRID].reshape(-1)


# speedup vs baseline: 1.0011x; 1.0011x over previous
import functools

import jax
import jax.numpy as jnp
from jax import lax
from jax.experimental import pallas as pl
from jax.experimental.pallas import tpu as pltpu
from jax.experimental.pallas import tpu_sc as plsc

_N = 10000
_E = 320000
_D = 128
_A = 8
_B = _N // _A

_NC = 2
_NS = 16
_NW = _NC * _NS
_EPW = _E // _NW
_CH = 128
_NCHUNK = _EPW // _CH
_CHT = _EPW - _NCHUNK * _CH
_TOFF = _NCHUNK * _CH
_SPAN = 624
_ZR = 208
_ZRA = 48

_RB = 2000
_GB = 256
_GRID = _N // _RB


def _sc_mesh():
    return plsc.VectorSubcoreMesh(core_axis_name="c", subcore_axis_name="s")


@functools.partial(
    pl.kernel,
    mesh=_sc_mesh(),
    out_type=jax.ShapeDtypeStruct((_NC, _N, 16), jnp.float32),
    scratch_types=[
        pltpu.VMEM((_CH,), jnp.int32),
        pltpu.VMEM((_CH,), jnp.int32),
        pltpu.VMEM((_CH,), jnp.int32),
        pltpu.VMEM((_CH,), jnp.int32),
        pltpu.VMEM((_CHT,), jnp.int32),
        pltpu.VMEM((_CH, 16), jnp.float32),
        pltpu.VMEM((_ZR, 16), jnp.float32),
        pltpu.VMEM_SHARED((_N, 16), jnp.float32),
        pltpu.SemaphoreType.DMA,
        pltpu.SemaphoreType.DMA,
        pltpu.SemaphoreType.DMA,
        pltpu.SemaphoreType.DMA,
        pltpu.SemaphoreType.DMA,
        pltpu.SemaphoreType.DMA,
        pltpu.SemaphoreType.DMA,
        pltpu.SemaphoreType.DMA,
    ],
)
def _deg_kernel(dst_hbm, degp_hbm, di0, di1, di2, di3, dit_v, ones_v,
                zero_v, deg_sh, d0, d1, d2, d3, s0, s1, s2, s3):
    c = lax.axis_index("c")
    s = lax.axis_index("s")
    wid = c * _NS + s
    didx = (di0, di1, di2, di3)
    dsem = (d0, d1, d2, d3)
    ssem = (s0, s1, s2, s3)
    base = wid * _EPW

    def fire_i(j, k):
        off = pl.multiple_of(base + j * _CH, 8)
        pltpu.async_copy(dst_hbm.at[pl.ds(off, _CH)], didx[k], dsem[k])

    def wait_i(j, k):
        off = pl.multiple_of(base + j * _CH, 8)
        pltpu.make_async_copy(dst_hbm.at[pl.ds(off, _CH)], didx[k],
                              dsem[k]).wait()

    for t in range(4):
        fire_i(t, t)

    one16 = jnp.ones((16,), jnp.float32)
    zro16 = jnp.zeros((16,), jnp.float32)

    def fill_ones(i, _):
        ones_v[i, :] = one16
        return _

    lax.fori_loop(0, _CH, fill_ones, None)

    def fill_zero(i, _):
        zero_v[i, :] = zro16
        return _

    lax.fori_loop(0, _ZR, fill_zero, None)

    span = pl.multiple_of(s * _SPAN, 8)
    for z in range(3):
        pltpu.sync_copy(zero_v, deg_sh.at[pl.ds(span + z * _ZR, _ZR)])

    @pl.when(s == _NS - 1)
    def _zero_tail():
        pltpu.sync_copy(zero_v.at[pl.ds(0, 16)],
                        deg_sh.at[pl.ds(_NS * _SPAN, 16)])

    plsc.subcore_barrier()

    def body(i, _):
        c0 = i * 4
        for t in range(4):
            j = c0 + t
            wait_i(j, t)
            pltpu.sync_copy(ones_v, deg_sh.at[didx[t]], add=True)

            @pl.when(j + 4 < _NCHUNK)
            def _pref():
                fire_i(j + 4, t)
        return _

    lax.fori_loop(0, _NCHUNK // 4, body, None)
    for t in range(2):
        j = (_NCHUNK // 4) * 4 + t
        wait_i(j, t)
        pltpu.sync_copy(ones_v, deg_sh.at[didx[t]], add=True)

    toff = pl.multiple_of(base + _TOFF, 8)
    pltpu.sync_copy(dst_hbm.at[pl.ds(toff, _CHT)], dit_v)
    pltpu.sync_copy(ones_v.at[pl.ds(0, _CHT)], deg_sh.at[dit_v], add=True)
    plsc.subcore_barrier()

    pltpu.sync_copy(deg_sh.at[pl.ds(span, _SPAN)],
                    degp_hbm.at[c, pl.ds(span, _SPAN)])

    @pl.when(s == _NS - 1)
    def _copy_tail():
        pltpu.sync_copy(deg_sh.at[pl.ds(_NS * _SPAN, 16)],
                        degp_hbm.at[c, pl.ds(_NS * _SPAN, 16)])


@functools.partial(
    pl.kernel,
    mesh=_sc_mesh(),
    out_type=jax.ShapeDtypeStruct((_NC, _N, _D), jnp.float32),
    scratch_types=[
        pltpu.VMEM((_CH,), jnp.int32),
        pltpu.VMEM((_CH,), jnp.int32),
        pltpu.VMEM((_CH,), jnp.int32),
        pltpu.VMEM((_CH,), jnp.int32),
        pltpu.VMEM((_CH,), jnp.int32),
        pltpu.VMEM((_CH,), jnp.int32),
        pltpu.VMEM((_CH,), jnp.int32),
        pltpu.VMEM((_CH,), jnp.int32),
        pltpu.VMEM((_CHT,), jnp.int32),
        pltpu.VMEM((_CHT,), jnp.int32),
        pltpu.VMEM((_CH, _D), jnp.float32),
        pltpu.VMEM((_CH, _D), jnp.float32),
        pltpu.VMEM_SHARED((_N, _D), jnp.float32),
        pltpu.SemaphoreType.DMA,
        pltpu.SemaphoreType.DMA,
        pltpu.SemaphoreType.DMA,
        pltpu.SemaphoreType.DMA,
        pltpu.SemaphoreType.DMA,
        pltpu.SemaphoreType.DMA,
        pltpu.SemaphoreType.DMA,
        pltpu.SemaphoreType.DMA,
        pltpu.SemaphoreType.DMA,
        pltpu.SemaphoreType.DMA,
    ],
)
def _agg_kernel(src_hbm, dst_hbm, g_hbm, aggp_hbm,
                si0, si1, si2, si3, di0, di1, di2, di3, sit_v, dit_v,
                r0, r1, agg_sh,
                g0, g1, i0, i1, i2, i3, d0, d1, d2, d3):
    c = lax.axis_index("c")
    s = lax.axis_index("s")
    wid = c * _NS + s
    rows = (r0, r1)
    sidx = (si0, si1, si2, si3)
    didx = (di0, di1, di2, di3)
    gsem = (g0, g1)
    isem = (i0, i1, i2, i3)
    dsem = (d0, d1, d2, d3)
    base = wid * _EPW

    zro16 = jnp.zeros((16,), jnp.float32)

    def fill_zero(i, _):
        r = i // 8
        k = i % 8
        r0[r, pl.ds(k * 16, 16)] = zro16
        return _

    lax.fori_loop(0, _CH * 8, fill_zero, None)

    span = pl.multiple_of(s * _SPAN, 8)
    for z in range(_SPAN // _CH):
        pltpu.sync_copy(r0, agg_sh.at[pl.ds(span + z * _CH, _CH)])
    pltpu.sync_copy(r0.at[pl.ds(0, _SPAN % _CH)],
                    agg_sh.at[pl.ds(span + (_SPAN // _CH) * _CH,
                                    _SPAN % _CH)])

    @pl.when(s == _NS - 1)
    def _zero_tail():
        pltpu.sync_copy(r0.at[pl.ds(0, 16)],
                        agg_sh.at[pl.ds(_NS * _SPAN, 16)])

    plsc.subcore_barrier()

    def fire_i(j, m):
        off = pl.multiple_of(base + j * _CH, 8)
        pltpu.async_copy(src_hbm.at[pl.ds(off, _CH)], sidx[m], isem[m])
        pltpu.async_copy(dst_hbm.at[pl.ds(off, _CH)], didx[m], dsem[m])

    def wait_i(j, m):
        off = pl.multiple_of(base + j * _CH, 8)
        pltpu.make_async_copy(src_hbm.at[pl.ds(off, _CH)], sidx[m],
                              isem[m]).wait()
        pltpu.make_async_copy(dst_hbm.at[pl.ds(off, _CH)], didx[m],
                              dsem[m]).wait()

    def fire_g(j, m, k):
        pltpu.async_copy(g_hbm.at[sidx[m]], rows[k], gsem[k])

    def wait_g(m, k):
        pltpu.make_async_copy(g_hbm.at[sidx[m]], rows[k],
                              gsem[k]).wait()

    def sync_s(j, m, k):
        pltpu.sync_copy(rows[k], agg_sh.at[didx[m]], add=True)

    for t in range(4):
        fire_i(t, t)
    for t in range(2):
        wait_i(t, t)
        fire_g(t, t, t)

    def body(i, _):
        c0 = i * 4
        for t in range(4):
            c = c0 + t
            k = t % 2
            wait_g(t, k)
            sync_s(c, t, k)

            @pl.when(c + 4 < _NCHUNK)
            def _pref_i():
                fire_i(c + 4, t)

            @pl.when(c + 2 < _NCHUNK)
            def _pref_g():
                m2 = (t + 2) % 4
                wait_i(c + 2, m2)
                fire_g(c + 2, m2, k)
        return _

    lax.fori_loop(0, _NCHUNK // 4, body, None)
    for t in range(2):
        c = (_NCHUNK // 4) * 4 + t
        wait_g(t, t)
        sync_s(c, t, t)

    toff = pl.multiple_of(base + _TOFF, 8)
    pltpu.sync_copy(src_hbm.at[pl.ds(toff, _CHT)], sit_v)
    pltpu.sync_copy(dst_hbm.at[pl.ds(toff, _CHT)], dit_v)
    pltpu.async_copy(g_hbm.at[sit_v], r0.at[pl.ds(0, _CHT)], g0).wait()
    pltpu.sync_copy(r0.at[pl.ds(0, _CHT)], agg_sh.at[dit_v], add=True)
    plsc.subcore_barrier()

    pltpu.sync_copy(agg_sh.at[pl.ds(span, _SPAN)],
                    aggp_hbm.at[c, pl.ds(span, _SPAN)])

    @pl.when(s == _NS - 1)
    def _copy_tail():
        pltpu.sync_copy(agg_sh.at[pl.ds(_NS * _SPAN, 16)],
                        aggp_hbm.at[c, pl.ds(_NS * _SPAN, 16)])


def _g_body(state_ref, w_ref, degp_ref, g_ref):
    dp = degp_ref[0] + degp_ref[1]
    dinv = lax.rsqrt(dp[:, 0:1] + 1.0)
    g_ref[...] = jnp.dot(state_ref[...] * dinv, w_ref[...],
                         preferred_element_type=jnp.float32)


def _g_call(state, w_conv, degp):
    return pl.pallas_call(
        _g_body,
        grid=(_GRID,),
        in_specs=[
            pl.BlockSpec((_RB, _D), lambda i: (i, 0)),
            pl.BlockSpec((_D, _D), lambda i: (0, 0)),
            pl.BlockSpec((_NC, _RB, 16), lambda i: (0, i, 0)),
        ],
        out_specs=pl.BlockSpec((_RB, _D), lambda i: (i, 0)),
        out_shape=jax.ShapeDtypeStruct((_N, _D), jnp.float32),
    )(state, w_conv, degp)


def _head_body(aggp_ref, g_ref, state_ref, degp_ref, acol_ref, bconv_ref,
               w1a_ref, w1b_ref, b1_ref, w2_ref, b2_ref, w3_ref, b3_ref,
               out_ref):
    agg = aggp_ref[0] + aggp_ref[1]
    dp = degp_ref[0] + degp_ref[1]
    dinv = lax.rsqrt(dp[:, 0:1] + 1.0)
    conv = dinv * (agg + g_ref[...]) + bconv_ref[...]
    x = jnp.maximum(conv, 0.0) + state_ref[...]
    y1 = jnp.dot(x, w1a_ref[...], preferred_element_type=jnp.float32)
    y1 = y1 + acol_ref[...] * w1b_ref[...] + b1_ref[...]
    y1 = jnp.maximum(y1, 0.0)
    y2 = jnp.dot(y1, w2_ref[...], preferred_element_type=jnp.float32)
    y2 = jnp.maximum(y2 + b2_ref[...], 0.0)
    rows = lax.broadcasted_iota(jnp.int32, (_GB, _RB), 0)
    cols = lax.broadcasted_iota(jnp.int32, (_GB, _RB), 1)
    sel = jnp.where((cols >> 3) == rows, 1.0, 0.0)
    grp = jnp.dot(sel, y2, preferred_element_type=jnp.float32)
    out_ref[...] = jnp.dot(grp, w3_ref[...],
                           preferred_element_type=jnp.float32) + b3_ref[...]


def _head_call(aggp, g, state, degp, acol, bconv, w1a, w1b, b1, w2, b2,
               w3p, b3p):
    full = lambda shape: pl.BlockSpec(shape, lambda i: tuple(0 for _ in shape))
    return pl.pallas_call(
        _head_body,
        grid=(_GRID,),
        in_specs=[
            pl.BlockSpec((_NC, _RB, _D), lambda i: (0, i, 0)),
            pl.BlockSpec((_RB, _D), lambda i: (i, 0)),
            pl.BlockSpec((_RB, _D), lambda i: (i, 0)),
            pl.BlockSpec((_NC, _RB, 16), lambda i: (0, i, 0)),
            pl.BlockSpec((_RB, 1), lambda i: (i, 0)),
            full((1, _D)),
            full((_D, 32)),
            full((1, 32)),
            full((1, 32)),
            full((32, 32)),
            full((1, 32)),
            full((32, _D)),
            full((1, _D)),
        ],
        out_specs=pl.BlockSpec((_GB, _D), lambda i: (i, 0)),
        out_shape=jax.ShapeDtypeStruct((_GRID * _GB, _D), jnp.float32),
    )(aggp, g, state, degp, acol, bconv, w1a, w1b, b1, w2, b2, w3p, b3p)


def kernel(state, edge_index, action, W_conv, b_conv, W1, b1, W2, b2, W3, b3):
    src = edge_index[0]
    dst = edge_index[1]

    degp = _deg_kernel(dst)
    g = _g_call(state, W_conv, degp)
    aggp = _agg_kernel(src, dst, g)

    acol = action.reshape(_N, 1)
    w1a = W1[:_D]
    w1b = W1[_D:_D + 1]
    w3p = jnp.pad(W3, ((0, 0), (0, _D - 1)))
    b3p = jnp.pad(b3.reshape(1, 1), ((0, 0), (0, _D - 1)))

    out = _head_call(aggp, g, state, degp, acol, b_conv.reshape(1, _D),
                     w1a, w1b, b1.reshape(1, 32), W2, b2.reshape(1, 32),
                     w3p, b3p)

    return out[:, 0].reshape(_GRID, _GB)[:, :_B // _GRID].reshape(-1)


# speedup vs baseline: 1.0036x; 1.0024x over previous
import functools

import jax
import jax.numpy as jnp
from jax import lax
from jax.experimental import pallas as pl
from jax.experimental.pallas import tpu as pltpu
from jax.experimental.pallas import tpu_sc as plsc

_N = 10000
_E = 320000
_D = 128
_A = 8
_B = _N // _A

_NC = 2
_NS = 16
_NW = _NC * _NS
_EPW = _E // _NW
_CH = 128
_NCHUNK = _EPW // _CH
_CHT = _EPW - _NCHUNK * _CH
_TOFF = _NCHUNK * _CH
_SPAN = 624
_ZR = 208

_RB = 2000
_GB = 256
_GRID = _N // _RB


def _sc_mesh():
    return plsc.VectorSubcoreMesh(core_axis_name="c", subcore_axis_name="s")


@functools.partial(
    pl.kernel,
    mesh=_sc_mesh(),
    out_type=jax.ShapeDtypeStruct((_NC, _N, 16), jnp.float32),
    scratch_types=[
        pltpu.VMEM((_CH,), jnp.int32),
        pltpu.VMEM((_CH,), jnp.int32),
        pltpu.VMEM((_CH,), jnp.int32),
        pltpu.VMEM((_CH,), jnp.int32),
        pltpu.VMEM((_CHT,), jnp.int32),
        pltpu.VMEM((_CH, 16), jnp.float32),
        pltpu.VMEM((_ZR, 16), jnp.float32),
        pltpu.VMEM_SHARED((_N, 16), jnp.float32),
        pltpu.SemaphoreType.DMA,
        pltpu.SemaphoreType.DMA,
        pltpu.SemaphoreType.DMA,
        pltpu.SemaphoreType.DMA,
    ],
)
def _deg_kernel(dst_hbm, degp_hbm, di0, di1, di2, di3, dit_v, ones_v,
                zero_v, deg_sh, d0, d1, d2, d3):
    c = lax.axis_index("c")
    s = lax.axis_index("s")
    wid = c * _NS + s
    didx = (di0, di1, di2, di3)
    dsem = (d0, d1, d2, d3)
    base = wid * _EPW

    def fire_i(j, k):
        off = pl.multiple_of(base + j * _CH, 8)
        pltpu.async_copy(dst_hbm.at[pl.ds(off, _CH)], didx[k], dsem[k])

    def wait_i(j, k):
        off = pl.multiple_of(base + j * _CH, 8)
        pltpu.make_async_copy(dst_hbm.at[pl.ds(off, _CH)], didx[k],
                              dsem[k]).wait()

    for t in range(4):
        fire_i(t, t)

    one16 = jnp.ones((16,), jnp.float32)
    zro16 = jnp.zeros((16,), jnp.float32)

    def fill_ones(i, _):
        ones_v[i, :] = one16
        return _

    lax.fori_loop(0, _CH, fill_ones, None)

    def fill_zero(i, _):
        zero_v[i, :] = zro16
        return _

    lax.fori_loop(0, _ZR, fill_zero, None)

    span = pl.multiple_of(s * _SPAN, 8)
    for z in range(3):
        pltpu.sync_copy(zero_v, deg_sh.at[pl.ds(span + z * _ZR, _ZR)])

    @pl.when(s == _NS - 1)
    def _zero_tail():
        pltpu.sync_copy(zero_v.at[pl.ds(0, 16)],
                        deg_sh.at[pl.ds(_NS * _SPAN, 16)])

    plsc.subcore_barrier()

    def body(i, _):
        c0 = i * 4
        for t in range(4):
            j = c0 + t
            wait_i(j, t)
            pltpu.sync_copy(ones_v, deg_sh.at[didx[t]], add=True)

            @pl.when(j + 4 < _NCHUNK)
            def _pref():
                fire_i(j + 4, t)
        return _

    lax.fori_loop(0, _NCHUNK // 4, body, None)
    for t in range(2):
        j = (_NCHUNK // 4) * 4 + t
        wait_i(j, t)
        pltpu.sync_copy(ones_v, deg_sh.at[didx[t]], add=True)

    toff = pl.multiple_of(base + _TOFF, 8)
    pltpu.sync_copy(dst_hbm.at[pl.ds(toff, _CHT)], dit_v)
    pltpu.sync_copy(ones_v.at[pl.ds(0, _CHT)], deg_sh.at[dit_v], add=True)
    plsc.subcore_barrier()

    pltpu.sync_copy(deg_sh.at[pl.ds(span, _SPAN)],
                    degp_hbm.at[c, pl.ds(span, _SPAN)])

    @pl.when(s == _NS - 1)
    def _copy_tail():
        pltpu.sync_copy(deg_sh.at[pl.ds(_NS * _SPAN, 16)],
                        degp_hbm.at[c, pl.ds(_NS * _SPAN, 16)])


@functools.partial(
    pl.kernel,
    mesh=_sc_mesh(),
    out_type=jax.ShapeDtypeStruct((_NC, _N, _D), jnp.float32),
    scratch_types=[
        pltpu.VMEM((_CH,), jnp.int32),
        pltpu.VMEM((_CH,), jnp.int32),
        pltpu.VMEM((_CH,), jnp.int32),
        pltpu.VMEM((_CH,), jnp.int32),
        pltpu.VMEM((_CH,), jnp.int32),
        pltpu.VMEM((_CH,), jnp.int32),
        pltpu.VMEM((_CH,), jnp.int32),
        pltpu.VMEM((_CH,), jnp.int32),
        pltpu.VMEM((_CHT,), jnp.int32),
        pltpu.VMEM((_CHT,), jnp.int32),
        pltpu.VMEM((_CH, _D), jnp.float32),
        pltpu.VMEM((_CH, _D), jnp.float32),
        pltpu.VMEM_SHARED((_N, _D), jnp.float32),
        pltpu.SemaphoreType.DMA,
        pltpu.SemaphoreType.DMA,
        pltpu.SemaphoreType.DMA,
        pltpu.SemaphoreType.DMA,
        pltpu.SemaphoreType.DMA,
        pltpu.SemaphoreType.DMA,
        pltpu.SemaphoreType.DMA,
        pltpu.SemaphoreType.DMA,
        pltpu.SemaphoreType.DMA,
        pltpu.SemaphoreType.DMA,
    ],
)
def _agg_kernel(src_hbm, dst_hbm, g_hbm, aggp_hbm,
                si0, si1, si2, si3, di0, di1, di2, di3, sit_v, dit_v,
                r0, r1, agg_sh,
                g0, g1, i0, i1, i2, i3, d0, d1, d2, d3):
    c = lax.axis_index("c")
    s = lax.axis_index("s")
    wid = c * _NS + s
    rows = (r0, r1)
    sidx = (si0, si1, si2, si3)
    didx = (di0, di1, di2, di3)
    gsem = (g0, g1)
    isem = (i0, i1, i2, i3)
    dsem = (d0, d1, d2, d3)
    base = wid * _EPW

    zro16 = jnp.zeros((16,), jnp.float32)

    def fill_zero(i, _):
        r = i // 8
        k = i % 8
        r0[r, pl.ds(k * 16, 16)] = zro16
        return _

    lax.fori_loop(0, _CH * 8, fill_zero, None)

    span = pl.multiple_of(s * _SPAN, 8)
    for z in range(_SPAN // _CH):
        pltpu.sync_copy(r0, agg_sh.at[pl.ds(span + z * _CH, _CH)])
    pltpu.sync_copy(r0.at[pl.ds(0, _SPAN % _CH)],
                    agg_sh.at[pl.ds(span + (_SPAN // _CH) * _CH,
                                    _SPAN % _CH)])

    @pl.when(s == _NS - 1)
    def _zero_tail():
        pltpu.sync_copy(r0.at[pl.ds(0, 16)],
                        agg_sh.at[pl.ds(_NS * _SPAN, 16)])

    plsc.subcore_barrier()

    def fire_i(j, m):
        off = pl.multiple_of(base + j * _CH, 8)
        pltpu.async_copy(src_hbm.at[pl.ds(off, _CH)], sidx[m], isem[m])
        pltpu.async_copy(dst_hbm.at[pl.ds(off, _CH)], didx[m], dsem[m])

    def wait_i(j, m):
        off = pl.multiple_of(base + j * _CH, 8)
        pltpu.make_async_copy(src_hbm.at[pl.ds(off, _CH)], sidx[m],
                              isem[m]).wait()
        pltpu.make_async_copy(dst_hbm.at[pl.ds(off, _CH)], didx[m],
                              dsem[m]).wait()

    def fire_g(j, m, k):
        pltpu.async_copy(g_hbm.at[sidx[m]], rows[k], gsem[k])

    def wait_g(m, k):
        pltpu.make_async_copy(g_hbm.at[sidx[m]], rows[k],
                              gsem[k]).wait()

    def sync_s(j, m, k):
        pltpu.sync_copy(rows[k], agg_sh.at[didx[m]], add=True)

    for t in range(4):
        fire_i(t, t)
    for t in range(2):
        wait_i(t, t)
        fire_g(t, t, t)

    def body(i, _):
        c0 = i * 4
        for t in range(4):
            c = c0 + t
            k = t % 2
            wait_g(t, k)
            sync_s(c, t, k)

            @pl.when(c + 4 < _NCHUNK)
            def _pref_i():
                fire_i(c + 4, t)

            @pl.when(c + 2 < _NCHUNK)
            def _pref_g():
                m2 = (t + 2) % 4
                wait_i(c + 2, m2)
                fire_g(c + 2, m2, k)
        return _

    lax.fori_loop(0, _NCHUNK // 4, body, None)
    for t in range(2):
        c = (_NCHUNK // 4) * 4 + t
        wait_g(t, t)
        sync_s(c, t, t)

    toff = pl.multiple_of(base + _TOFF, 8)
    pltpu.sync_copy(src_hbm.at[pl.ds(toff, _CHT)], sit_v)
    pltpu.sync_copy(dst_hbm.at[pl.ds(toff, _CHT)], dit_v)
    pltpu.async_copy(g_hbm.at[sit_v], r0.at[pl.ds(0, _CHT)], g0).wait()
    pltpu.sync_copy(r0.at[pl.ds(0, _CHT)], agg_sh.at[dit_v], add=True)
    plsc.subcore_barrier()

    pltpu.sync_copy(agg_sh.at[pl.ds(span, _SPAN)],
                    aggp_hbm.at[c, pl.ds(span, _SPAN)])

    @pl.when(s == _NS - 1)
    def _copy_tail():
        pltpu.sync_copy(agg_sh.at[pl.ds(_NS * _SPAN, 16)],
                        aggp_hbm.at[c, pl.ds(_NS * _SPAN, 16)])


def _g_body(state_ref, w_ref, degp_ref, g_ref):
    dp = degp_ref[0] + degp_ref[1]
    dinv = lax.rsqrt(dp[:, 0:1] + 1.0)
    g_ref[...] = jnp.dot(state_ref[...] * dinv, w_ref[...],
                         preferred_element_type=jnp.float32)


def _g_call(state, w_conv, degp):
    return pl.pallas_call(
        _g_body,
        grid=(_GRID,),
        in_specs=[
            pl.BlockSpec((_RB, _D), lambda i: (i, 0)),
            pl.BlockSpec((_D, _D), lambda i: (0, 0)),
            pl.BlockSpec((_NC, _RB, 16), lambda i: (0, i, 0)),
        ],
        out_specs=pl.BlockSpec((_RB, _D), lambda i: (i, 0)),
        out_shape=jax.ShapeDtypeStruct((_N, _D), jnp.float32),
    )(state, w_conv, degp)


def _head_body(aggp_ref, g_ref, state_ref, degp_ref, acol_ref, bconv_ref,
               w1a_ref, w1b_ref, b1_ref, w2_ref, b2_ref, w3_ref, b3_ref,
               out_ref):
    agg = aggp_ref[0] + aggp_ref[1]
    dp = degp_ref[0] + degp_ref[1]
    dinv = lax.rsqrt(dp[:, 0:1] + 1.0)
    conv = dinv * (agg + g_ref[...]) + bconv_ref[...]
    x = jnp.maximum(conv, 0.0) + state_ref[...]
    y1 = jnp.dot(x, w1a_ref[...], preferred_element_type=jnp.float32)
    y1 = y1 + acol_ref[...] * w1b_ref[...] + b1_ref[...]
    y1 = jnp.maximum(y1, 0.0)
    y2 = jnp.dot(y1, w2_ref[...], preferred_element_type=jnp.float32)
    y2 = jnp.maximum(y2 + b2_ref[...], 0.0)
    rows = lax.broadcasted_iota(jnp.int32, (_GB, _RB), 0)
    cols = lax.broadcasted_iota(jnp.int32, (_GB, _RB), 1)
    sel = jnp.where((cols >> 3) == rows, 1.0, 0.0)
    grp = jnp.dot(sel, y2, preferred_element_type=jnp.float32)
    out_ref[...] = jnp.dot(grp, w3_ref[...],
                           preferred_element_type=jnp.float32) + b3_ref[...]


def _head_call(aggp, g, state, degp, acol, bconv, w1a, w1b, b1, w2, b2,
               w3p, b3p):
    full = lambda shape: pl.BlockSpec(shape, lambda i: tuple(0 for _ in shape))
    return pl.pallas_call(
        _head_body,
        grid=(_GRID,),
        in_specs=[
            pl.BlockSpec((_NC, _RB, _D), lambda i: (0, i, 0)),
            pl.BlockSpec((_RB, _D), lambda i: (i, 0)),
            pl.BlockSpec((_RB, _D), lambda i: (i, 0)),
            pl.BlockSpec((_NC, _RB, 16), lambda i: (0, i, 0)),
            pl.BlockSpec((_RB, 1), lambda i: (i, 0)),
            full((1, _D)),
            full((_D, 32)),
            full((1, 32)),
            full((1, 32)),
            full((32, 32)),
            full((1, 32)),
            full((32, _D)),
            full((1, _D)),
        ],
        out_specs=pl.BlockSpec((_GB, _D), lambda i: (i, 0)),
        out_shape=jax.ShapeDtypeStruct((_GRID * _GB, _D), jnp.float32),
    )(aggp, g, state, degp, acol, bconv, w1a, w1b, b1, w2, b2, w3p, b3p)


def kernel(state, edge_index, action, W_conv, b_conv, W1, b1, W2, b2, W3, b3):
    src = edge_index[0]
    dst = edge_index[1]

    degp = _deg_kernel(dst)
    g = _g_call(state, W_conv, degp)
    aggp = _agg_kernel(src, dst, g)

    acol = action.reshape(_N, 1)
    w1a = W1[:_D]
    w1b = W1[_D:_D + 1]
    w3p = jnp.pad(W3, ((0, 0), (0, _D - 1)))
    b3p = jnp.pad(b3.reshape(1, 1), ((0, 0), (0, _D - 1)))

    out = _head_call(aggp, g, state, degp, acol, b_conv.reshape(1, _D),
                     w1a, w1b, b1.reshape(1, 32), W2, b2.reshape(1, 32),
                     w3p, b3p)

    return out[:, 0].reshape(_GRID, _GB)[:, :_B // _GRID].reshape(-1)
